# baseline (speedup 1.0000x reference)
import math

import jax
import jax.numpy as jnp
from jax.experimental import pallas as pl
from jax.experimental.pallas import tpu as pltpu

NUM_CLASSES = 3
IMG_SIZE = 640.0
LAMBDA_BOX = 7.5
LAMBDA_CLS = 0.5
TOPK = 10
ALPHA_COST = 0.5
BETA_COST = 6.0
GAMMA = 2.0
ALPHA = 0.25
EPS = 1e-6
CLASS_WEIGHTS = (1.1757211179195934, 0.09527723808100434, 1.7290016439994023)

N_ANCH = 4800
N_GT = 20
N_IMG = 8
NROW = N_IMG * 24
NC = N_GT * TOPK
S = 40
STRIDE = IMG_SIZE / S
BIG = 1e30


def _sigmoid(x):
    return 1.0 / (1.0 + jnp.exp(-x))


def _softplus(x):
    return jnp.maximum(x, 0.0) + jnp.log1p(jnp.exp(-jnp.abs(x)))


def _atan_pos(r):
    z = jnp.minimum(r, 1.0 / r)
    t = z / (1.0 + jnp.sqrt(1.0 + z * z))
    t2 = t * t
    p = t * (1.0 + t2 * (-1.0 / 3.0 + t2 * (1.0 / 5.0 + t2 * (-1.0 / 7.0
         + t2 * (1.0 / 9.0 + t2 * (-1.0 / 11.0 + t2 * (1.0 / 13.0
         + t2 * (-1.0 / 15.0))))))))
    a = 2.0 * p
    return jnp.where(r <= 1.0, a, math.pi / 2.0 - a)


def _loss_kernel(p_ref, tb_ref, lab_ref, out_ref, vals_ref):
    TX = p_ref[0]
    TY = p_ref[1]
    TW = p_ref[2]
    TH = p_ref[3]
    L0 = p_ref[4]
    L1 = p_ref[5]
    L2 = p_ref[6]

    lane8 = jax.lax.broadcasted_iota(jnp.int32, (N_IMG, N_ANCH), 1)
    gx = (lane8 % S).astype(jnp.float32)
    gy = ((lane8 // S) % S).astype(jnp.float32)

    PX = (_sigmoid(TX) * 2.0 - 0.5 + gx) * STRIDE
    PY = (_sigmoid(TY) * 2.0 - 0.5 + gy) * STRIDE
    PW = _softplus(TW) * STRIDE
    PH = _softplus(TH) * STRIDE
    PX1 = PX - 0.5 * PW
    PY1 = PY - 0.5 * PH
    PX2 = PX + 0.5 * PW
    PY2 = PY + 0.5 * PH
    SG0 = _sigmoid(L0)
    SG1 = _sigmoid(L1)
    SG2 = _sigmoid(L2)

    lane = jax.lax.broadcasted_iota(jnp.int32, (1, N_ANCH), 1)
    lane_f32 = lane.astype(jnp.float32)
    dn = (((1,), (1,)), ((), ()))
    e7 = 1e-7

    costs = []
    tcoords = []
    labcols = []
    for b in range(N_IMG):
        t = tb_ref[b] * IMG_SIZE
        tx1 = t[:, 0:1]
        ty1 = t[:, 1:2]
        tx2 = t[:, 2:3]
        ty2 = t[:, 3:4]
        tcoords.append((tx1, ty1, tx2, ty2))
        labcol = lab_ref[b].reshape(N_GT, 1)
        labcols.append(labcol)

        px1 = PX1[b:b + 1, :]
        py1 = PY1[b:b + 1, :]
        px2 = PX2[b:b + 1, :]
        py2 = PY2[b:b + 1, :]

        p_sel = ((labcol == 0).astype(jnp.float32) * SG0[b:b + 1, :]
                 + (labcol == 1).astype(jnp.float32) * SG1[b:b + 1, :]
                 + (labcol == 2).astype(jnp.float32) * SG2[b:b + 1, :])
        cost_cls = -jnp.log(jnp.clip(p_sel, EPS, 1.0 - EPS))

        parea = jnp.maximum(px2 - px1, 0.0) * jnp.maximum(py2 - py1, 0.0)
        tarea = jnp.maximum(tx2 - tx1, 0.0) * jnp.maximum(ty2 - ty1, 0.0)
        inter = (jnp.maximum(jnp.minimum(px2, tx2) - jnp.maximum(px1, tx1), 0.0)
                 * jnp.maximum(jnp.minimum(py2, ty2) - jnp.maximum(py1, ty1), 0.0))
        union = parea + tarea - inter + e7
        iou = jnp.clip(inter / union, 0.0, 1.0)

        cost_b = ALPHA_COST * cost_cls + BETA_COST * (1.0 - iou)
        costs.append(cost_b)

    vals_ref[...] = jnp.full((NROW, N_ANCH), BIG, dtype=jnp.float32)
    for b in range(N_IMG):
        vals_ref[24 * b:24 * b + 20, :] = costs[b]

    nidxs = []
    for _ in range(TOPK):
        vals = vals_ref[...]
        m = jnp.min(vals, axis=1, keepdims=True)
        nidx = jnp.min(jnp.where(vals <= m, lane, N_ANCH),
                       axis=1, keepdims=True)
        nidxs.append(nidx)
        vals_ref[...] = vals + (lane == nidx).astype(jnp.float32) * BIG

    subk = [[] for _ in range(N_IMG)]
    candk = [[] for _ in range(N_IMG)]
    for b in range(N_IMG):
        rows = [n[24 * b:24 * b + 20, :] for n in nidxs]
        nidxcat = jnp.concatenate(rows, axis=0)
        SELb = (lane == nidxcat).astype(jnp.float32)
        subk[b].append(jax.lax.dot_general(
            costs[b], SELb, dn, precision=jax.lax.Precision.HIGHEST,
            preferred_element_type=jnp.float32))
        for r in rows:
            candk[b].append(jnp.transpose(r).astype(jnp.float32))

    planes = [jnp.concatenate(subk[b], axis=1).reshape(1, N_GT, NC)
              for b in range(N_IMG)]
    candrows = [jnp.concatenate(candk[b], axis=1).reshape(1, 1, NC)
                for b in range(N_IMG)]
    A = jnp.concatenate(planes, axis=0)
    candf = jnp.concatenate(candrows, axis=0)

    giota = jax.lax.broadcasted_iota(jnp.int32, (1, N_GT, 1), 1)
    ciota = jax.lax.broadcasted_iota(jnp.int32, (1, 1, NC), 2)
    gl20 = jax.lax.broadcasted_iota(jnp.int32, (N_IMG, N_GT), 1)
    ai = jnp.full((N_IMG, N_GT), -1, dtype=jnp.int32)
    for _ in range(N_GT):
        m = jnp.min(jnp.min(A, axis=2, keepdims=True), axis=1,
                    keepdims=True)
        ok = m < BIG * 0.5
        enc = jnp.where((A <= m) & ok, giota * 2048 + ciota, 1 << 20)
        idx = jnp.min(jnp.min(enc, axis=2, keepdims=True), axis=1,
                      keepdims=True)
        gstar = idx // 2048
        cstar = idx - gstar * 2048
        astar = jnp.min(jnp.where(ciota == cstar, candf, BIG), axis=2,
                        keepdims=True)
        ai = jnp.where((gl20 == jnp.squeeze(gstar, axis=2))
                       & jnp.squeeze(ok, axis=2),
                       jnp.squeeze(astar, axis=2).astype(jnp.int32), ai)
        A = (A + ((giota == gstar) & ok).astype(jnp.float32) * BIG
             + ((candf == astar) & ok).astype(jnp.float32) * BIG)

    loss_box = jnp.zeros((), dtype=jnp.float32)
    loss_cls = jnp.zeros((), dtype=jnp.float32)
    npos = jnp.zeros((), dtype=jnp.float32)
    for b in range(N_IMG):
        aib = ai[b:b + 1, :].reshape(N_GT, 1)
        tx1, ty1, tx2, ty2 = tcoords[b]
        labcol = labcols[b]
        validg = aib >= 0
        validf = validg.astype(jnp.float32)
        aidx = jnp.where(validg, aib, 0)

        onehot = (lane == aidx).astype(jnp.float32)
        pbx1 = jnp.sum(onehot * PX1[b:b + 1, :], axis=1, keepdims=True)
        pby1 = jnp.sum(onehot * PY1[b:b + 1, :], axis=1, keepdims=True)
        pbx2 = jnp.sum(onehot * PX2[b:b + 1, :], axis=1, keepdims=True)
        pby2 = jnp.sum(onehot * PY2[b:b + 1, :], axis=1, keepdims=True)

        pwg = jnp.maximum(pbx2 - pbx1, e7)
        phg = jnp.maximum(pby2 - pby1, e7)
        twg = jnp.maximum(tx2 - tx1, e7)
        thg = jnp.maximum(ty2 - ty1, e7)
        iw = jnp.maximum(jnp.minimum(pbx2, tx2) - jnp.maximum(pbx1, tx1), 0.0)
        ih = jnp.maximum(jnp.minimum(pby2, ty2) - jnp.maximum(pby1, ty1), 0.0)
        inter_g = iw * ih
        union_g = pwg * phg + twg * thg - inter_g + e7
        iou_g = inter_g / union_g
        cw = jnp.maximum(pbx2, tx2) - jnp.minimum(pbx1, tx1)
        ch = jnp.maximum(pby2, ty2) - jnp.minimum(pby1, ty1)
        c2 = cw * cw + ch * ch + e7
        rho2 = ((pbx1 + pbx2 - tx1 - tx2) ** 2
                + (pby1 + pby2 - ty1 - ty2) ** 2) / 4.0
        datan = _atan_pos(twg / thg) - _atan_pos(pwg / phg)
        v = (4.0 / (math.pi ** 2)) * datan * datan
        alpha_t = v / (v - iou_g + 1.0 + e7)
        ciou = iou_g - rho2 / c2 - alpha_t * v
        loss_box = loss_box + jnp.sum((1.0 - ciou) * validf)
        npos = npos + jnp.sum(validf)

        for c, lc, sg in ((0, L0[b:b + 1, :], SG0[b:b + 1, :]),
                          (1, L1[b:b + 1, :], SG1[b:b + 1, :]),
                          (2, L2[b:b + 1, :], SG2[b:b + 1, :])):
            wc = validf * (labcol == c).astype(jnp.float32)
            tcls = jnp.minimum(jnp.sum(onehot * wc, axis=0, keepdims=True), 1.0)
            ce = (jnp.maximum(lc, 0.0) - lc * tcls
                  + jnp.log1p(jnp.exp(-jnp.abs(lc))))
            p_t = sg * tcls + (1.0 - sg) * (1.0 - tcls)
            fl = ce * (1.0 - p_t) ** GAMMA
            alpha_w = ALPHA * tcls + (1.0 - ALPHA) * (1.0 - tcls)
            loss_cls = loss_cls + CLASS_WEIGHTS[c] * jnp.sum(alpha_w * fl)

    lane128 = jax.lax.broadcasted_iota(jnp.int32, (1, 128), 1)
    outvec = (jnp.where(lane128 == 0, loss_box, 0.0)
              + jnp.where(lane128 == 1, loss_cls, 0.0)
              + jnp.where(lane128 == 2, npos, 0.0))
    out_ref[...] = outvec


@jax.jit
def kernel(preds, targets_boxes, targets_labels):
    B = preds.shape[0]
    p = preds.reshape(B, N_ANCH, 7).transpose(2, 0, 1)
    tb = targets_boxes.astype(jnp.float32)
    lab = targets_labels.astype(jnp.int32).reshape(B, 1, N_GT)

    out = pl.pallas_call(
        _loss_kernel,
        out_shape=jax.ShapeDtypeStruct((1, 128), jnp.float32),
        scratch_shapes=[pltpu.VMEM((NROW, N_ANCH), jnp.float32)],
    )(p, tb, lab)

    loss_box = out[0, 0]
    loss_cls = out[0, 1]
    npos = out[0, 2]
    denom = jnp.maximum(npos, 1.0)
    return (LAMBDA_BOX * loss_box + LAMBDA_CLS * loss_cls) / denom


# speedup vs baseline: 1.0250x; 1.0250x over previous
import math

import jax
import jax.numpy as jnp
from jax.experimental import pallas as pl
from jax.experimental.pallas import tpu as pltpu

NUM_CLASSES = 3
IMG_SIZE = 640.0
LAMBDA_BOX = 7.5
LAMBDA_CLS = 0.5
TOPK = 10
ALPHA_COST = 0.5
BETA_COST = 6.0
GAMMA = 2.0
ALPHA = 0.25
EPS = 1e-6
CLASS_WEIGHTS = (1.1757211179195934, 0.09527723808100434, 1.7290016439994023)

N_ANCH = 4800
N_GT = 20
N_IMG = 8
NROW = N_IMG * 24
NC = N_GT * TOPK
S = 40
STRIDE = IMG_SIZE / S
BIG = 1e30


def _sigmoid(x):
    return 1.0 / (1.0 + jnp.exp(-x))


def _softplus(x):
    return jnp.maximum(x, 0.0) + jnp.log1p(jnp.exp(-jnp.abs(x)))


def _atan_pos(r):
    z = jnp.minimum(r, 1.0 / r)
    t = z / (1.0 + jnp.sqrt(1.0 + z * z))
    t2 = t * t
    p = t * (1.0 + t2 * (-1.0 / 3.0 + t2 * (1.0 / 5.0 + t2 * (-1.0 / 7.0
         + t2 * (1.0 / 9.0 + t2 * (-1.0 / 11.0 + t2 * (1.0 / 13.0
         + t2 * (-1.0 / 15.0))))))))
    a = 2.0 * p
    return jnp.where(r <= 1.0, a, math.pi / 2.0 - a)


def _loss_kernel(p_ref, tb_ref, lab_ref, out_ref, vals_ref):
    TX = p_ref[0]
    TY = p_ref[1]
    TW = p_ref[2]
    TH = p_ref[3]
    L0 = p_ref[4]
    L1 = p_ref[5]
    L2 = p_ref[6]

    lane8 = jax.lax.broadcasted_iota(jnp.int32, (N_IMG, N_ANCH), 1)
    gx = (lane8 % S).astype(jnp.float32)
    gy = ((lane8 // S) % S).astype(jnp.float32)

    PX = (_sigmoid(TX) * 2.0 - 0.5 + gx) * STRIDE
    PY = (_sigmoid(TY) * 2.0 - 0.5 + gy) * STRIDE
    PW = _softplus(TW) * STRIDE
    PH = _softplus(TH) * STRIDE
    PX1 = PX - 0.5 * PW
    PY1 = PY - 0.5 * PH
    PX2 = PX + 0.5 * PW
    PY2 = PY + 0.5 * PH
    SG0 = _sigmoid(L0)
    SG1 = _sigmoid(L1)
    SG2 = _sigmoid(L2)

    lane = jax.lax.broadcasted_iota(jnp.int32, (1, N_ANCH), 1)
    lane_f32 = lane.astype(jnp.float32)
    dn = (((1,), (1,)), ((), ()))
    e7 = 1e-7

    costs = []
    tcoords = []
    labcols = []
    for b in range(N_IMG):
        t = tb_ref[b] * IMG_SIZE
        tx1 = t[:, 0:1]
        ty1 = t[:, 1:2]
        tx2 = t[:, 2:3]
        ty2 = t[:, 3:4]
        tcoords.append((tx1, ty1, tx2, ty2))
        labcol = lab_ref[b].reshape(N_GT, 1)
        labcols.append(labcol)

        px1 = PX1[b:b + 1, :]
        py1 = PY1[b:b + 1, :]
        px2 = PX2[b:b + 1, :]
        py2 = PY2[b:b + 1, :]

        p_sel = ((labcol == 0).astype(jnp.float32) * SG0[b:b + 1, :]
                 + (labcol == 1).astype(jnp.float32) * SG1[b:b + 1, :]
                 + (labcol == 2).astype(jnp.float32) * SG2[b:b + 1, :])
        cost_cls = -jnp.log(jnp.clip(p_sel, EPS, 1.0 - EPS))

        parea = jnp.maximum(px2 - px1, 0.0) * jnp.maximum(py2 - py1, 0.0)
        tarea = jnp.maximum(tx2 - tx1, 0.0) * jnp.maximum(ty2 - ty1, 0.0)
        inter = (jnp.maximum(jnp.minimum(px2, tx2) - jnp.maximum(px1, tx1), 0.0)
                 * jnp.maximum(jnp.minimum(py2, ty2) - jnp.maximum(py1, ty1), 0.0))
        union = parea + tarea - inter + e7
        iou = jnp.clip(inter / union, 0.0, 1.0)

        cost_b = ALPHA_COST * cost_cls + BETA_COST * (1.0 - iou)
        costs.append(cost_b)

    vals_ref[...] = jnp.full((NROW, N_ANCH), BIG, dtype=jnp.float32)
    for b in range(N_IMG):
        vals_ref[24 * b:24 * b + 20, :] = costs[b]

    subk = [[] for _ in range(N_IMG)]
    candk = [[] for _ in range(N_IMG)]
    for _ in range(TOPK // 2):
        v0 = vals_ref[...]
        m = jnp.min(v0, axis=1, keepdims=True)
        nidxA = jnp.min(jnp.where(v0 <= m, lane, N_ANCH),
                        axis=1, keepdims=True)
        selA = (lane == nidxA).astype(jnp.float32)
        v1 = v0 + selA * BIG
        m = jnp.min(v1, axis=1, keepdims=True)
        nidxB = jnp.min(jnp.where(v1 <= m, lane, N_ANCH),
                        axis=1, keepdims=True)
        selB = (lane == nidxB).astype(jnp.float32)
        vals_ref[...] = v1 + selB * BIG
        for b in range(N_IMG):
            vb = v0[24 * b:24 * b + 20, :]
            sb = jnp.concatenate(
                [selA[24 * b:24 * b + 20, :], selB[24 * b:24 * b + 20, :]],
                axis=0)
            subk[b].append(jax.lax.dot_general(
                vb, sb, dn, precision=jax.lax.Precision.HIGHEST,
                preferred_element_type=jnp.float32))
            candk[b].append(jnp.transpose(
                nidxA[24 * b:24 * b + 20, :]).astype(jnp.float32))
            candk[b].append(jnp.transpose(
                nidxB[24 * b:24 * b + 20, :]).astype(jnp.float32))

    planes = [jnp.concatenate(subk[b], axis=1).reshape(1, N_GT, NC)
              for b in range(N_IMG)]
    candrows = [jnp.concatenate(candk[b], axis=1).reshape(1, 1, NC)
                for b in range(N_IMG)]
    A = jnp.concatenate(planes, axis=0)
    candf = jnp.concatenate(candrows, axis=0)

    giota = jax.lax.broadcasted_iota(jnp.int32, (1, N_GT, 1), 1)
    ciota = jax.lax.broadcasted_iota(jnp.int32, (1, 1, NC), 2)
    gl20 = jax.lax.broadcasted_iota(jnp.int32, (N_IMG, N_GT), 1)
    ai = jnp.full((N_IMG, N_GT), -1, dtype=jnp.int32)
    for _ in range(N_GT):
        m = jnp.min(jnp.min(A, axis=2, keepdims=True), axis=1,
                    keepdims=True)
        ok = m < BIG * 0.5
        enc = jnp.where((A <= m) & ok, giota * 2048 + ciota, 1 << 20)
        idx = jnp.min(jnp.min(enc, axis=2, keepdims=True), axis=1,
                      keepdims=True)
        gstar = idx // 2048
        cstar = idx - gstar * 2048
        astar = jnp.min(jnp.where(ciota == cstar, candf, BIG), axis=2,
                        keepdims=True)
        ai = jnp.where((gl20 == jnp.squeeze(gstar, axis=2))
                       & jnp.squeeze(ok, axis=2),
                       jnp.squeeze(astar, axis=2).astype(jnp.int32), ai)
        A = (A + ((giota == gstar) & ok).astype(jnp.float32) * BIG
             + ((candf == astar) & ok).astype(jnp.float32) * BIG)

    loss_box = jnp.zeros((), dtype=jnp.float32)
    loss_cls = jnp.zeros((), dtype=jnp.float32)
    npos = jnp.zeros((), dtype=jnp.float32)
    for b in range(N_IMG):
        aib = ai[b:b + 1, :].reshape(N_GT, 1)
        tx1, ty1, tx2, ty2 = tcoords[b]
        labcol = labcols[b]
        validg = aib >= 0
        validf = validg.astype(jnp.float32)
        aidx = jnp.where(validg, aib, 0)

        onehot = (lane == aidx).astype(jnp.float32)
        pbx1 = jnp.sum(onehot * PX1[b:b + 1, :], axis=1, keepdims=True)
        pby1 = jnp.sum(onehot * PY1[b:b + 1, :], axis=1, keepdims=True)
        pbx2 = jnp.sum(onehot * PX2[b:b + 1, :], axis=1, keepdims=True)
        pby2 = jnp.sum(onehot * PY2[b:b + 1, :], axis=1, keepdims=True)

        pwg = jnp.maximum(pbx2 - pbx1, e7)
        phg = jnp.maximum(pby2 - pby1, e7)
        twg = jnp.maximum(tx2 - tx1, e7)
        thg = jnp.maximum(ty2 - ty1, e7)
        iw = jnp.maximum(jnp.minimum(pbx2, tx2) - jnp.maximum(pbx1, tx1), 0.0)
        ih = jnp.maximum(jnp.minimum(pby2, ty2) - jnp.maximum(pby1, ty1), 0.0)
        inter_g = iw * ih
        union_g = pwg * phg + twg * thg - inter_g + e7
        iou_g = inter_g / union_g
        cw = jnp.maximum(pbx2, tx2) - jnp.minimum(pbx1, tx1)
        ch = jnp.maximum(pby2, ty2) - jnp.minimum(pby1, ty1)
        c2 = cw * cw + ch * ch + e7
        rho2 = ((pbx1 + pbx2 - tx1 - tx2) ** 2
                + (pby1 + pby2 - ty1 - ty2) ** 2) / 4.0
        datan = _atan_pos(twg / thg) - _atan_pos(pwg / phg)
        v = (4.0 / (math.pi ** 2)) * datan * datan
        alpha_t = v / (v - iou_g + 1.0 + e7)
        ciou = iou_g - rho2 / c2 - alpha_t * v
        loss_box = loss_box + jnp.sum((1.0 - ciou) * validf)
        npos = npos + jnp.sum(validf)

        for c, lc, sg in ((0, L0[b:b + 1, :], SG0[b:b + 1, :]),
                          (1, L1[b:b + 1, :], SG1[b:b + 1, :]),
                          (2, L2[b:b + 1, :], SG2[b:b + 1, :])):
            wc = validf * (labcol == c).astype(jnp.float32)
            tcls = jnp.minimum(jnp.sum(onehot * wc, axis=0, keepdims=True), 1.0)
            ce = (jnp.maximum(lc, 0.0) - lc * tcls
                  + jnp.log1p(jnp.exp(-jnp.abs(lc))))
            p_t = sg * tcls + (1.0 - sg) * (1.0 - tcls)
            fl = ce * (1.0 - p_t) ** GAMMA
            alpha_w = ALPHA * tcls + (1.0 - ALPHA) * (1.0 - tcls)
            loss_cls = loss_cls + CLASS_WEIGHTS[c] * jnp.sum(alpha_w * fl)

    lane128 = jax.lax.broadcasted_iota(jnp.int32, (1, 128), 1)
    outvec = (jnp.where(lane128 == 0, loss_box, 0.0)
              + jnp.where(lane128 == 1, loss_cls, 0.0)
              + jnp.where(lane128 == 2, npos, 0.0))
    out_ref[...] = outvec


@jax.jit
def kernel(preds, targets_boxes, targets_labels):
    B = preds.shape[0]
    p = preds.reshape(B, N_ANCH, 7).transpose(2, 0, 1)
    tb = targets_boxes.astype(jnp.float32)
    lab = targets_labels.astype(jnp.int32).reshape(B, 1, N_GT)

    out = pl.pallas_call(
        _loss_kernel,
        out_shape=jax.ShapeDtypeStruct((1, 128), jnp.float32),
        scratch_shapes=[pltpu.VMEM((NROW, N_ANCH), jnp.float32)],
    )(p, tb, lab)

    loss_box = out[0, 0]
    loss_cls = out[0, 1]
    npos = out[0, 2]
    denom = jnp.maximum(npos, 1.0)
    return (LAMBDA_BOX * loss_box + LAMBDA_CLS * loss_cls) / denom
